# baseline (speedup 1.0000x reference)
import jax
import jax.numpy as jnp
from jax import lax
from jax.experimental import pallas as pl
from jax.experimental.pallas import tpu as pltpu

P_MASK = 0.5
TIME_MASKING_PARA = 100
TIME_MASK_NUM = 1


def _mask_bounds(B, T):
    key = jax.random.key(42)
    key, k_apply = jax.random.split(key)
    apply_mask = jax.random.uniform(k_apply) <= P_MASK
    starts_l, ends_l = [], []
    for _ in range(TIME_MASK_NUM):
        key, k_t, k_t0 = jax.random.split(key, 3)
        t = jax.random.randint(k_t, (), 0, TIME_MASKING_PARA + 1)
        t0s = jax.random.randint(k_t0, (B,), 0, T - TIME_MASKING_PARA)
        t_eff = jnp.where(apply_mask, t, 0)
        starts_l.append(t0s.astype(jnp.int32))
        ends_l.append((t0s + t_eff).astype(jnp.int32))
    return starts_l[0], ends_l[0]


def _body(starts_ref, ends_ref, x_ref, o_ref):
    b = pl.program_id(0)
    s = starts_ref[b]
    e = ends_ref[b]
    col = lax.broadcasted_iota(jnp.int32, x_ref.shape, 1)
    o_ref[...] = jnp.where((col >= s) & (col < e), jnp.float32(0.0), x_ref[...])


def kernel(mel_batch):
    B, F, T = mel_batch.shape
    starts, ends = _mask_bounds(B, T)
    x2d = mel_batch.reshape(B * F, T)
    out = pl.pallas_call(
        _body,
        grid_spec=pltpu.PrefetchScalarGridSpec(
            num_scalar_prefetch=2,
            grid=(B,),
            in_specs=[pl.BlockSpec((F, T), lambda b, s, e: (b, 0))],
            out_specs=pl.BlockSpec((F, T), lambda b, s, e: (b, 0)),
        ),
        out_shape=jax.ShapeDtypeStruct((B * F, T), jnp.float32),
    )(starts, ends, x2d)
    return out.reshape(B, F, T)
